# baseline (speedup 1.0000x reference)
import jax
import jax.numpy as jnp
from jax.experimental import pallas as pl
from jax.experimental.pallas import tpu as pltpu

_N = 4096
_D = 64
_BM = 256
_NCHUNK = _N // _BM
_NBUF = 4


def _mm_kernel(adj_hbm, emb_ref, out_ref, *scratch):
    bufs = scratch[:_NBUF]
    sems = scratch[_NBUF:]

    def copy(i):
        s = i % _NBUF
        return pltpu.make_async_copy(
            adj_hbm.at[pl.ds(i * _BM, _BM), :], bufs[s], sems[s]
        )

    for i in range(_NBUF):
        copy(i).start()
    for i in range(_NCHUNK):
        copy(i).wait()
        out_ref[pl.ds(i * _BM, _BM), :] = jnp.dot(
            bufs[i % _NBUF][...], emb_ref[...],
            preferred_element_type=jnp.float32,
        )
        if i + _NBUF < _NCHUNK:
            copy(i + _NBUF).start()


def kernel(adj, embeds):
    return pl.pallas_call(
        _mm_kernel,
        in_specs=[
            pl.BlockSpec(memory_space=pl.ANY),
            pl.BlockSpec(memory_space=pltpu.MemorySpace.VMEM),
        ],
        out_specs=pl.BlockSpec(memory_space=pltpu.MemorySpace.VMEM),
        out_shape=jax.ShapeDtypeStruct((_N, _D), jnp.float32),
        scratch_shapes=(
            [pltpu.VMEM((_BM, _N), jnp.float32) for _ in range(_NBUF)]
            + [pltpu.SemaphoreType.DMA for _ in range(_NBUF)]
        ),
    )(adj, embeds)


# speedup vs baseline: 1.0265x; 1.0265x over previous
import jax
import jax.numpy as jnp
from jax.experimental import pallas as pl
from jax.experimental.pallas import tpu as pltpu

_N = 4096
_D = 64
_BM = 128
_NCHUNK = _N // _BM
_NBUF = 8


def _mm_kernel(adj_hbm, emb_ref, out_ref, *scratch):
    bufs = scratch[:_NBUF]
    sems = scratch[_NBUF:]

    def copy(i):
        s = i % _NBUF
        return pltpu.make_async_copy(
            adj_hbm.at[pl.ds(i * _BM, _BM), :], bufs[s], sems[s]
        )

    for i in range(_NBUF):
        copy(i).start()
    for i in range(_NCHUNK):
        copy(i).wait()
        out_ref[pl.ds(i * _BM, _BM), :] = jnp.dot(
            bufs[i % _NBUF][...], emb_ref[...],
            preferred_element_type=jnp.float32,
        )
        if i + _NBUF < _NCHUNK:
            copy(i + _NBUF).start()


def kernel(adj, embeds):
    return pl.pallas_call(
        _mm_kernel,
        in_specs=[
            pl.BlockSpec(memory_space=pl.ANY),
            pl.BlockSpec(memory_space=pltpu.MemorySpace.VMEM),
        ],
        out_specs=pl.BlockSpec(memory_space=pltpu.MemorySpace.VMEM),
        out_shape=jax.ShapeDtypeStruct((_N, _D), jnp.float32),
        scratch_shapes=(
            [pltpu.VMEM((_BM, _N), jnp.float32) for _ in range(_NBUF)]
            + [pltpu.SemaphoreType.DMA for _ in range(_NBUF)]
        ),
    )(adj, embeds)
